# baseline (speedup 1.0000x reference)
import functools

import jax
import jax.numpy as jnp
from jax import lax
from jax.experimental import pallas as pl
from jax.experimental.pallas import tpu as pltpu
from jax.experimental.pallas import tpu_sc as plsc

N = 10000
E = 320000
NW = 32
EPW = E // NW
K = 40
SB = 25
NSB = EPW // (SB * K)
EG = 10
RING = 4


@functools.cache
def _mesh():
    return plsc.VectorSubcoreMesh(core_axis_name="c", subcore_axis_name="s",
                                  num_cores=2, num_subcores=16)


_SC_PARAMS = pltpu.CompilerParams(needs_layout_passes=False)


def _deg_body(dst_hbm, w_hbm, out_hbm, dst_v, w_v, deg_v):
    c = lax.axis_index("c")
    s = lax.axis_index("s")
    wid = c * 16 + s
    pltpu.sync_copy(dst_hbm.at[wid], dst_v)
    pltpu.sync_copy(w_hbm.at[wid], w_v)

    zeros = jnp.zeros((16,), jnp.float32)

    def zero_body(i, carry):
        deg_v[pl.ds(i * 16, 16)] = zeros
        return carry

    lax.fori_loop(0, N // 16, zero_body, 0)

    def edge_body(i, carry):
        idx = dst_v[pl.ds(i * 16, 16)]
        wv = w_v[pl.ds(i * 16, 16)]
        plsc.addupdate_scatter(deg_v, [idx], wv)
        return carry

    lax.fori_loop(0, EPW // 16, edge_body, 0)
    pltpu.sync_copy(deg_v, out_hbm.at[wid])


@functools.cache
def _make_deg():
    return functools.partial(
        pl.kernel,
        mesh=_mesh(),
        out_type=jax.ShapeDtypeStruct((NW, N), jnp.float32),
        scratch_types=[
            pltpu.VMEM((EPW,), jnp.int32),
            pltpu.VMEM((EPW,), jnp.float32),
            pltpu.VMEM((N,), jnp.float32),
        ],
        compiler_params=_SC_PARAMS,
    )(_deg_body)


@functools.cache
def _make_agg(F):
    nscale = F // 16

    @functools.partial(
        pl.kernel,
        mesh=_mesh(),
        out_type=jax.ShapeDtypeStruct((2, N, 128), jnp.float32),
        scratch_types=[
            pltpu.VMEM((SB, K), jnp.int32),
            pltpu.VMEM((SB, K), jnp.int32),
            pltpu.VMEM((SB, K), jnp.float32),
        ] + [pltpu.VMEM((K, 128), jnp.float32)] * RING + [
            pltpu.VMEM_SHARED((N, 128), jnp.float32),
        ] + [pltpu.SemaphoreType.DMA] * (2 * RING),
        compiler_params=_SC_PARAMS,
    )
    def _agg(h_hbm, src_hbm, dst_hbm, w_hbm, zero_hbm, out_hbm,
             src_b, dst_b, w_b, *rest):
        rows = list(rest[:RING])
        acc_sh = rest[RING]
        semg = list(rest[RING + 1:RING + 1 + RING])
        sems = list(rest[RING + 1 + RING:])
        c = lax.axis_index("c")
        s = lax.axis_index("s")
        wid = c * 16 + s
        row0 = pl.multiple_of(s * 632, 8)

        @pl.when(s < 15)
        def _zero_main():
            pltpu.sync_copy(zero_hbm.at[pl.ds(row0, 632)],
                            acc_sh.at[pl.ds(row0, 632)])

        @pl.when(s == 15)
        def _zero_tail():
            pltpu.sync_copy(zero_hbm.at[pl.ds(9480, 520)],
                            acc_sh.at[pl.ds(9480, 520)])

        plsc.subcore_barrier()

        def g_issue(j, b):
            pltpu.async_copy(h_hbm.at[src_b.at[j]], rows[b], semg[b])

        def g_wait(j, b):
            pltpu.make_async_copy(h_hbm.at[src_b.at[j]], rows[b],
                                  semg[b]).wait()

        def s_issue(j, b):
            pltpu.async_copy(rows[b], acc_sh.at[dst_b.at[j]], sems[b],
                             add=True)

        def s_wait(j, b):
            pltpu.make_async_copy(rows[b], acc_sh.at[dst_b.at[j]],
                                  sems[b]).wait()

        def scale(jj, b):
            rows_v = rows[b]
            idx_j = jnp.full((16,), jj, dtype=jnp.int32)

            def group(eg, carry):
                for i in range(EG):
                    idx_e = jnp.full((16,), i, dtype=jnp.int32) + eg * EG
                    wsp = plsc.load_gather(w_b, [idx_j, idx_e])
                    e0 = eg * EG + i
                    for k in range(nscale):
                        rows_v[e0, pl.ds(k * 16, 16)] = (
                            rows_v[e0, pl.ds(k * 16, 16)] * wsp)
                return carry

            lax.fori_loop(0, K // EG, group, 0)

        def super_chunk(sb, carry):
            pltpu.sync_copy(src_hbm.at[wid, sb], src_b)
            pltpu.sync_copy(dst_hbm.at[wid, sb], dst_b)
            pltpu.sync_copy(w_hbm.at[wid, sb], w_b)
            g_issue(0, 0)
            g_issue(1, 1)
            g_issue(2, 2)
            g_wait(0, 0)
            scale(0, 0)
            s_issue(0, 0)
            g_issue(3, 3)

            def quad(jt, carry2):
                j0 = 1 + jt * 4
                for q in range(4):
                    j = j0 + q
                    b = (1 + q) % RING
                    g_wait(j, b)
                    scale(j, b)
                    s_issue(j, b)
                    s_wait(j - 1, q)
                    g_issue(j + 3, q)
                return carry2

            lax.fori_loop(0, 5, quad, carry)
            for j in (21, 22, 23, 24):
                b = j % RING
                g_wait(j, b)
                scale(j, b)
                s_issue(j, b)
                if j == 21:
                    s_wait(20, 0)
                    g_issue(24, 0)
            for j in (21, 22, 23, 24):
                s_wait(j, j % RING)
            return carry

        lax.fori_loop(0, NSB, super_chunk, 0)
        plsc.subcore_barrier()

        @pl.when(s < 15)
        def _out_main():
            pltpu.sync_copy(acc_sh.at[pl.ds(row0, 632)],
                            out_hbm.at[c, pl.ds(row0, 632)])

        @pl.when(s == 15)
        def _out_tail():
            pltpu.sync_copy(acc_sh.at[pl.ds(9480, 520)],
                            out_hbm.at[c, pl.ds(9480, 520)])

    return _agg


def _bn(v, g, b, eps=1e-5):
    mu = jnp.mean(v, axis=0)
    var = jnp.var(v, axis=0)
    return (v - mu) / jnp.sqrt(var + eps) * g + b


def _tc_call(body, out_shapes, *args):
    return pl.pallas_call(body, out_shape=out_shapes)(*args)


def _stage1_body(x_ref, degp_ref, g_ref, b_ref, w_ref, hs_ref, dis_ref):
    deg = jnp.sum(degp_ref[...], axis=0) + 1.0
    dis = lax.rsqrt(deg)
    a = _bn(x_ref[...], g_ref[...], b_ref[...])
    h = jnp.dot(a, w_ref[...], preferred_element_type=jnp.float32)
    hs_ref[...] = h * dis[:, None]
    dis_ref[...] = dis[:, None]


def _stage_mid_body(p0_ref, p1_ref, hs_ref, dis_ref, bias_ref, g_ref, b_ref,
                    w_ref, out_ref):
    dis = dis_ref[...]
    agg = dis * (p0_ref[...] + p1_ref[...] + hs_ref[...]) + bias_ref[...]
    a = jax.nn.relu(_bn(agg, g_ref[...], b_ref[...]))
    h = jnp.dot(a, w_ref[...], preferred_element_type=jnp.float32)
    out_ref[...] = h * dis


def _stage_final_body(p0_ref, p1_ref, hs_ref, dis_ref, bias_ref, g5_ref,
                      b5_ref, l1w_ref, l1b_ref, g6_ref, b6_ref, l2w_ref,
                      l2b_ref, out_ref):
    dis = dis_ref[...]
    agg = dis * (p0_ref[...] + p1_ref[...] + hs_ref[...]) + bias_ref[...]
    h = _bn(agg, g5_ref[...], b5_ref[...])
    h = jnp.dot(jax.nn.relu(h), l1w_ref[...],
                preferred_element_type=jnp.float32) + l1b_ref[...]
    h = _bn(h, g6_ref[...], b6_ref[...])
    out_ref[...] = jnp.dot(jax.nn.relu(h), l2w_ref[...],
                           preferred_element_type=jnp.float32) + l2b_ref[...]


def _padv(v):
    return jnp.pad(v, (0, 128 - v.shape[0]))


def _padm(m, cols=128):
    return jnp.pad(m, ((0, 128 - m.shape[0]), (0, cols - m.shape[1])))


def kernel(x, edge_weight, params, edge_index):
    p = params
    src4 = edge_index[0].reshape(NW, NSB, SB, K)
    dst4 = edge_index[1].reshape(NW, NSB, SB, K)
    w4 = edge_weight.reshape(NW, NSB, SB, K)
    dst2 = edge_index[1].reshape(NW, EPW)
    w2 = edge_weight.reshape(NW, EPW)
    zeros = jnp.zeros((N, 128), jnp.float32)

    deg_parts = _make_deg()(dst2, w2)

    hs1, dis = _tc_call(
        _stage1_body,
        (jax.ShapeDtypeStruct((N, 128), jnp.float32),
         jax.ShapeDtypeStruct((N, 1), jnp.float32)),
        x, deg_parts, p['bn1_g'], p['bn1_b'], p['W1'])

    def conv_step(hs, F, bias, g, b, w_next):
        parts = _make_agg(F)(hs, src4, dst4, w4, zeros)
        return _tc_call(
            _stage_mid_body,
            jax.ShapeDtypeStruct((N, 128), jnp.float32),
            parts[0], parts[1], hs, dis,
            _padv(bias), _padv(g), _padv(b), _padm(w_next))

    hs2 = conv_step(hs1, 128, p['b1'], p['bn2_g'], p['bn2_b'], p['W2'])
    hs3 = conv_step(hs2, 128, p['b2'], p['bn3_g'], p['bn3_b'], p['W3'])
    hs4 = conv_step(hs3, 64, p['b3'], p['bn4_g'], p['bn4_b'], p['W4'])

    parts4 = _make_agg(32)(hs4, src4, dst4, w4, zeros)
    out = _tc_call(
        _stage_final_body,
        jax.ShapeDtypeStruct((N, 40), jnp.float32),
        parts4[0], parts4[1], hs4, dis, _padv(p['b4']),
        _padv(p['bn5_g']), _padv(p['bn5_b']),
        _padm(p['lin1_W']), _padv(p['lin1_b']),
        _padv(p['bn6_g']), _padv(p['bn6_b']),
        _padm(p['lin2_W'], cols=40), p['lin2_b'])
    return out


# speedup vs baseline: 1.5276x; 1.5276x over previous
import functools

import jax
import jax.numpy as jnp
from jax import lax
from jax.experimental import pallas as pl
from jax.experimental.pallas import tpu as pltpu
from jax.experimental.pallas import tpu_sc as plsc

N = 10000
E = 320000
NW = 32
EPW = E // NW
K = 80
SB = 25
NSB = EPW // (SB * K)
EG = 16
RING = 3


@functools.cache
def _mesh():
    return plsc.VectorSubcoreMesh(core_axis_name="c", subcore_axis_name="s",
                                  num_cores=2, num_subcores=16)


_SC_PARAMS = pltpu.CompilerParams(needs_layout_passes=False)


def _deg_body(dst_hbm, w_hbm, out_hbm, dst_v, w_v, deg_v):
    c = lax.axis_index("c")
    s = lax.axis_index("s")
    wid = c * 16 + s
    pltpu.sync_copy(dst_hbm.at[wid], dst_v)
    pltpu.sync_copy(w_hbm.at[wid], w_v)

    zeros = jnp.zeros((16,), jnp.float32)

    def zero_body(i, carry):
        deg_v[pl.ds(i * 16, 16)] = zeros
        return carry

    lax.fori_loop(0, N // 16, zero_body, 0)

    def edge_body(i, carry):
        idx = dst_v[pl.ds(i * 16, 16)]
        wv = w_v[pl.ds(i * 16, 16)]
        plsc.addupdate_scatter(deg_v, [idx], wv)
        return carry

    lax.fori_loop(0, EPW // 16, edge_body, 0)
    pltpu.sync_copy(deg_v, out_hbm.at[wid])


@functools.cache
def _make_deg():
    return functools.partial(
        pl.kernel,
        mesh=_mesh(),
        out_type=jax.ShapeDtypeStruct((NW, N), jnp.float32),
        scratch_types=[
            pltpu.VMEM((EPW,), jnp.int32),
            pltpu.VMEM((EPW,), jnp.float32),
            pltpu.VMEM((N,), jnp.float32),
        ],
        compiler_params=_SC_PARAMS,
    )(_deg_body)


@functools.cache
def _make_agg(F):
    nscale = F // 16

    @functools.partial(
        pl.kernel,
        mesh=_mesh(),
        out_type=jax.ShapeDtypeStruct((2, N, 128), jnp.float32),
        scratch_types=[
            pltpu.VMEM((SB, K), jnp.int32),
            pltpu.VMEM((SB, K), jnp.int32),
            pltpu.VMEM((SB, K), jnp.float32),
        ] + [pltpu.VMEM((K, 128), jnp.float32)] * RING + [
            pltpu.VMEM_SHARED((N, 128), jnp.float32),
        ] + [pltpu.SemaphoreType.DMA] * (2 * RING),
        compiler_params=_SC_PARAMS,
    )
    def _agg(h_hbm, src_hbm, dst_hbm, w_hbm, zero_hbm, out_hbm,
             src_b, dst_b, w_b, *rest):
        rows = list(rest[:RING])
        acc_sh = rest[RING]
        semg = list(rest[RING + 1:RING + 1 + RING])
        sems = list(rest[RING + 1 + RING:])
        c = lax.axis_index("c")
        s = lax.axis_index("s")
        wid = c * 16 + s
        row0 = pl.multiple_of(s * 632, 8)

        @pl.when(s < 15)
        def _zero_main():
            pltpu.sync_copy(zero_hbm.at[pl.ds(row0, 632)],
                            acc_sh.at[pl.ds(row0, 632)])

        @pl.when(s == 15)
        def _zero_tail():
            pltpu.sync_copy(zero_hbm.at[pl.ds(9480, 520)],
                            acc_sh.at[pl.ds(9480, 520)])

        plsc.subcore_barrier()

        def g_issue(j, b):
            pltpu.async_copy(h_hbm.at[src_b.at[j]], rows[b], semg[b])

        def g_wait(j, b):
            pltpu.make_async_copy(h_hbm.at[src_b.at[j]], rows[b],
                                  semg[b]).wait()

        def s_issue(j, b):
            pltpu.async_copy(rows[b], acc_sh.at[dst_b.at[j]], sems[b],
                             add=True)

        def s_wait(j, b):
            pltpu.make_async_copy(rows[b], acc_sh.at[dst_b.at[j]],
                                  sems[b]).wait()

        def scale(jj, b):
            rows_v = rows[b]
            idx_j = jnp.full((16,), jj, dtype=jnp.int32)

            def group(eg, carry):
                for i in range(EG):
                    idx_e = jnp.full((16,), i, dtype=jnp.int32) + eg * EG
                    wsp = plsc.load_gather(w_b, [idx_j, idx_e])
                    e0 = eg * EG + i
                    for k in range(nscale):
                        rows_v[e0, pl.ds(k * 16, 16)] = (
                            rows_v[e0, pl.ds(k * 16, 16)] * wsp)
                return carry

            lax.fori_loop(0, K // EG, group, 0)

        def super_chunk(sb, carry):
            pltpu.sync_copy(src_hbm.at[wid, sb], src_b)
            pltpu.sync_copy(dst_hbm.at[wid, sb], dst_b)
            pltpu.sync_copy(w_hbm.at[wid, sb], w_b)
            g_issue(0, 0)
            g_issue(1, 1)
            g_wait(0, 0)
            scale(0, 0)
            s_issue(0, 0)
            g_issue(2, 2)

            def triple(jt, carry2):
                j0 = 1 + jt * 3
                for q in range(3):
                    j = j0 + q
                    b = (1 + q) % RING
                    g_wait(j, b)
                    scale(j, b)
                    s_issue(j, b)
                    s_wait(j - 1, q)
                    g_issue(j + 2, q)
                return carry2

            lax.fori_loop(0, 7, triple, carry)
            for j in (22, 23, 24):
                b = j % RING
                g_wait(j, b)
                scale(j, b)
                s_issue(j, b)
                if j == 22:
                    s_wait(21, 0)
                    g_issue(24, 0)
            for j in (22, 23, 24):
                s_wait(j, j % RING)
            return carry

        lax.fori_loop(0, NSB, super_chunk, 0)
        plsc.subcore_barrier()

        @pl.when(s < 15)
        def _out_main():
            pltpu.sync_copy(acc_sh.at[pl.ds(row0, 632)],
                            out_hbm.at[c, pl.ds(row0, 632)])

        @pl.when(s == 15)
        def _out_tail():
            pltpu.sync_copy(acc_sh.at[pl.ds(9480, 520)],
                            out_hbm.at[c, pl.ds(9480, 520)])

    return _agg


def _bn(v, g, b, eps=1e-5):
    mu = jnp.mean(v, axis=0)
    var = jnp.var(v, axis=0)
    return (v - mu) / jnp.sqrt(var + eps) * g + b


def _tc_call(body, out_shapes, *args):
    return pl.pallas_call(body, out_shape=out_shapes)(*args)


def _stage1_body(x_ref, degp_ref, g_ref, b_ref, w_ref, hs_ref, dis_ref):
    deg = jnp.sum(degp_ref[...], axis=0) + 1.0
    dis = lax.rsqrt(deg)
    a = _bn(x_ref[...], g_ref[...], b_ref[...])
    h = jnp.dot(a, w_ref[...], preferred_element_type=jnp.float32)
    hs_ref[...] = h * dis[:, None]
    dis_ref[...] = dis[:, None]


def _stage_mid_body(p0_ref, p1_ref, hs_ref, dis_ref, bias_ref, g_ref, b_ref,
                    w_ref, out_ref):
    dis = dis_ref[...]
    agg = dis * (p0_ref[...] + p1_ref[...] + hs_ref[...]) + bias_ref[...]
    a = jax.nn.relu(_bn(agg, g_ref[...], b_ref[...]))
    h = jnp.dot(a, w_ref[...], preferred_element_type=jnp.float32)
    out_ref[...] = h * dis


def _stage_final_body(p0_ref, p1_ref, hs_ref, dis_ref, bias_ref, g5_ref,
                      b5_ref, l1w_ref, l1b_ref, g6_ref, b6_ref, l2w_ref,
                      l2b_ref, out_ref):
    dis = dis_ref[...]
    agg = dis * (p0_ref[...] + p1_ref[...] + hs_ref[...]) + bias_ref[...]
    h = _bn(agg, g5_ref[...], b5_ref[...])
    h = jnp.dot(jax.nn.relu(h), l1w_ref[...],
                preferred_element_type=jnp.float32) + l1b_ref[...]
    h = _bn(h, g6_ref[...], b6_ref[...])
    out_ref[...] = jnp.dot(jax.nn.relu(h), l2w_ref[...],
                           preferred_element_type=jnp.float32) + l2b_ref[...]


def _padv(v):
    return jnp.pad(v, (0, 128 - v.shape[0]))


def _padm(m, cols=128):
    return jnp.pad(m, ((0, 128 - m.shape[0]), (0, cols - m.shape[1])))


def kernel(x, edge_weight, params, edge_index):
    p = params
    src4 = edge_index[0].reshape(NW, NSB, SB, K)
    dst4 = edge_index[1].reshape(NW, NSB, SB, K)
    w4 = edge_weight.reshape(NW, NSB, SB, K)
    dst2 = edge_index[1].reshape(NW, EPW)
    w2 = edge_weight.reshape(NW, EPW)
    zeros = jnp.zeros((N, 128), jnp.float32)

    deg_parts = _make_deg()(dst2, w2)

    hs1, dis = _tc_call(
        _stage1_body,
        (jax.ShapeDtypeStruct((N, 128), jnp.float32),
         jax.ShapeDtypeStruct((N, 1), jnp.float32)),
        x, deg_parts, p['bn1_g'], p['bn1_b'], p['W1'])

    def conv_step(hs, F, bias, g, b, w_next):
        parts = _make_agg(F)(hs, src4, dst4, w4, zeros)
        return _tc_call(
            _stage_mid_body,
            jax.ShapeDtypeStruct((N, 128), jnp.float32),
            parts[0], parts[1], hs, dis,
            _padv(bias), _padv(g), _padv(b), _padm(w_next))

    hs2 = conv_step(hs1, 128, p['b1'], p['bn2_g'], p['bn2_b'], p['W2'])
    hs3 = conv_step(hs2, 128, p['b2'], p['bn3_g'], p['bn3_b'], p['W3'])
    hs4 = conv_step(hs3, 64, p['b3'], p['bn4_g'], p['bn4_b'], p['W4'])

    parts4 = _make_agg(32)(hs4, src4, dst4, w4, zeros)
    out = _tc_call(
        _stage_final_body,
        jax.ShapeDtypeStruct((N, 40), jnp.float32),
        parts4[0], parts4[1], hs4, dis, _padv(p['b4']),
        _padv(p['bn5_g']), _padv(p['bn5_b']),
        _padm(p['lin1_W']), _padv(p['lin1_b']),
        _padv(p['bn6_g']), _padv(p['bn6_b']),
        _padm(p['lin2_W'], cols=40), p['lin2_b'])
    return out
